# baseline (speedup 1.0000x reference)
import functools

import jax
import jax.numpy as jnp
from jax import lax
from jax.experimental import pallas as pl
from jax.experimental.pallas import tpu as pltpu
from jax.experimental.pallas import tpu_sc as plsc

NUM_HASH = 16
BLOCK_SIZE = 256
SAMPLE_SIZE = 256
_HB = 2

_NC = 2
_NS = 16
_NW = _NC * _NS

_H = 12
_S = 8192
_D = 64
_DP = 128
_ROWS = _H * _S
_RPW = _ROWS // _NW
_CH = 128
_NCH = _RPW // _CH


def _sc_mesh():
    return plsc.VectorSubcoreMesh(core_axis_name="c", subcore_axis_name="s")


def _wid():
    return lax.axis_index("s") * _NC + lax.axis_index("c")


def _hash_body(q_ref, k_ref, r_ref, qh_ref, kh_ref):
    R = r_ref[...]
    w = 2 ** lax.broadcasted_iota(jnp.int32, (1, NUM_HASH), 1)

    pq = jax.lax.dot_general(q_ref[0], R, (((1,), (0,)), ((), ())),
                             preferred_element_type=jnp.float32)
    pk = jax.lax.dot_general(k_ref[0], R, (((1,), (0,)), ((), ())),
                             preferred_element_type=jnp.float32)
    qh_ref[...] = jnp.sum(jnp.where(pq > 0, w, 0), axis=-1).astype(
        jnp.float32).reshape(_S // _CH, _CH)
    kh_ref[...] = jnp.sum(jnp.where(pk > 0, w, 0), axis=-1).astype(
        jnp.float32).reshape(_S // _CH, _CH)


def _hash(q0, k0, R):
    out_types = (
        jax.ShapeDtypeStruct((_ROWS // _CH, _CH), jnp.float32),
        jax.ShapeDtypeStruct((_ROWS // _CH, _CH), jnp.float32),
    )
    return pl.pallas_call(
        _hash_body,
        grid=(_H,),
        in_specs=[
            pl.BlockSpec((1, _S, _D), lambda h: (h, 0, 0)),
            pl.BlockSpec((1, _S, _D), lambda h: (h, 0, 0)),
            pl.BlockSpec((_D, NUM_HASH), lambda h: (0, 0)),
        ],
        out_specs=[
            pl.BlockSpec((_S // _CH, _CH), lambda h: (h, 0)),
            pl.BlockSpec((_S // _CH, _CH), lambda h: (h, 0)),
        ],
        out_shape=out_types,
    )(q0, k0, R)


def _pack_body(q_ref, k_ref, v_ref, qpad_ref, kv_ref):
    qb = q_ref[0]
    qpad_ref[0, :, :_D] = qb
    qpad_ref[0, :, _D:] = jnp.zeros_like(qb)
    kv_ref[0, :, :_D] = k_ref[0]
    kv_ref[0, :, _D:] = v_ref[0]


def _pack(q0, k0, v0):
    out_types = (
        jax.ShapeDtypeStruct((_H, _S, _DP), jnp.float32),
        jax.ShapeDtypeStruct((_H, _S, _DP), jnp.float32),
    )
    qpad, kv = pl.pallas_call(
        _pack_body,
        grid=(_H,),
        in_specs=[
            pl.BlockSpec((1, _S, _D), lambda h: (h, 0, 0)),
            pl.BlockSpec((1, _S, _D), lambda h: (h, 0, 0)),
            pl.BlockSpec((1, _S, _D), lambda h: (h, 0, 0)),
        ],
        out_specs=[
            pl.BlockSpec((1, _S, _DP), lambda h: (h, 0, 0)),
            pl.BlockSpec((1, _S, _DP), lambda h: (h, 0, 0)),
        ],
        out_shape=out_types,
    )(q0, k0, v0)
    return qpad.reshape(_ROWS, _DP), kv.reshape(_ROWS, _DP)


_NCODES = 1 << NUM_HASH
_NSORT = 2 * _H


_L = 16
_CPL = _S // _L


def _lg(ref, addr):
    if len(ref.shape) == 2:
        v = plsc.load_gather(ref, [addr >> 7, addr & 127])
        return v.astype(jnp.int32) if v.dtype == jnp.float32 else v
    return plsc.load_gather(ref, [addr])


def _ss(ref, addr, val):
    if ref.dtype == jnp.float32 and val.dtype == jnp.int32:
        val = plsc.bitcast(val, jnp.float32)
    if len(ref.shape) == 2:
        plsc.store_scatter(ref, [addr >> 7, addr & 127], val)
    else:
        plsc.store_scatter(ref, [addr], val)


def _sort_codes(qh, kh):
    @functools.partial(
        pl.kernel,
        out_type=jax.ShapeDtypeStruct((2 * _ROWS // _CH, _CH), jnp.float32),
        scratch_types=[
            pltpu.VMEM((_S // _CH, _CH), jnp.float32),
            pltpu.VMEM((_S,), jnp.int32),
            pltpu.VMEM((_S,), jnp.int32),
            pltpu.VMEM((_S // _CH, _CH), jnp.float32),
            pltpu.VMEM((256 * _L,), jnp.int32),
        ],
        mesh=_sc_mesh(),
        compiler_params=pltpu.CompilerParams(needs_layout_passes=False),
    )
    def k(qh_hbm, kh_hbm, perm_hbm, c0, c1, v1, v2, hist):
        w = _wid()

        @pl.when(w < _H)
        def _():
            pltpu.sync_copy(qh_hbm.at[pl.ds(w * (_S // _CH), _S // _CH)], c0)

        @pl.when((w >= _H) & (w < _NSORT))
        def _():
            pltpu.sync_copy(
                kh_hbm.at[pl.ds((w - _H) * (_S // _CH), _S // _CH)], c0)

        @pl.when(w < _NSORT)
        def _():
            lane = jax.lax.iota(jnp.int32, 16)
            zeros = jnp.zeros((16,), jnp.int32)

            def radix_pass(src_c, src_v, dst_c, dst_v, shift, base):
                def zb(j, c):
                    hist[pl.ds(j * 16, 16)] = zeros
                    return c
                lax.fori_loop(0, 256, zb, 0)

                def hb(j, c):
                    addr = lane * _CPL + j
                    cv = _lg(src_c, addr)
                    digit = (cv >> shift) & 255
                    haddr = digit * _L + lane
                    cur = plsc.load_gather(hist, [haddr])
                    plsc.store_scatter(hist, [haddr], cur + 1)
                    return c
                lax.fori_loop(0, _CPL, hb, 0)

                def sb(j, carry):
                    vec = hist[pl.ds(j * 16, 16)]
                    total = jnp.sum(vec)
                    hist[pl.ds(j * 16, 16)] = plsc.cumsum(vec) - vec + carry
                    return carry + total
                lax.fori_loop(0, 256, sb, jnp.int32(0))

                def pb(j, c):
                    addr = lane * _CPL + j
                    cv = _lg(src_c, addr)
                    vv = addr if src_v is None else plsc.load_gather(
                        src_v, [addr])
                    digit = (cv >> shift) & 255
                    haddr = digit * _L + lane
                    pos = plsc.load_gather(hist, [haddr])
                    plsc.store_scatter(hist, [haddr], pos + 1)
                    if dst_c is not None:
                        plsc.store_scatter(dst_c, [pos], cv)
                    _ss(dst_v, pos, vv + base)
                    return c
                lax.fori_loop(0, _CPL, pb, 0)

            radix_pass(c0, None, c1, v1, 0, 0)
            radix_pass(c1, v1, None, v2, 8, (w % _H) * _S)
            row0 = jnp.where(w < _H, w * (_S // _CH),
                             _ROWS // _CH + (w - _H) * (_S // _CH))
            pltpu.sync_copy(v2, perm_hbm.at[pl.ds(row0, _S // _CH)])

    return k(qh, kh)


_NSLOT = 4
_NGRP = _NCH // _NSLOT


def _bitcast_rows(src_f, dst_i):
    lane = jax.lax.iota(jnp.int32, 16)

    def body(t, c):
        a = t * 16 + lane
        v = plsc.load_gather(src_f, [a >> 7, a & 127])
        plsc.store_scatter(dst_i, [a >> 7, a & 127],
                           plsc.bitcast(v, jnp.int32))
        return c
    lax.fori_loop(0, _NCH * _CH // 16, body, 0)


def _gather_rows(qpad, kv, perm2d):
    out_t = jax.ShapeDtypeStruct((_ROWS, _DP), jnp.float32)

    @functools.partial(
        pl.kernel,
        out_type=(out_t, out_t),
        scratch_types=[
            pltpu.VMEM((_NCH, _CH), jnp.int32),
            pltpu.VMEM((_NCH, _CH), jnp.int32),
            pltpu.VMEM((_NCH, _CH), jnp.float32),
            pltpu.VMEM((_NSLOT, _CH, _DP), jnp.float32),
        ] + [pltpu.SemaphoreType.DMA] * (2 * _NSLOT),
        mesh=_sc_mesh(),
        compiler_params=pltpu.CompilerParams(needs_layout_passes=False),
    )
    def k(qf, kvf, perm, qs, kvs, idxq_v, idxk_v, idx_f, rows4, *sems):
        gsems, wsems = sems[:_NSLOT], sems[_NSLOT:]
        w = _wid()
        pltpu.sync_copy(perm.at[pl.ds(w * _NCH, _NCH)], idx_f)
        _bitcast_rows(idx_f, idxq_v)
        pltpu.sync_copy(
            perm.at[pl.ds(_ROWS // _CH + w * _NCH, _NCH)], idx_f)
        _bitcast_rows(idx_f, idxk_v)

        def run_table(tab, idx_v, out):
            def gstart(j, t):
                pltpu.async_copy(tab.at[idx_v.at[j]], rows4.at[t], gsems[t])

            def gwait(j, t):
                pltpu.make_async_copy(
                    tab.at[idx_v.at[j]], rows4.at[t], gsems[t]).wait()

            def wslice(j):
                return out.at[pl.ds(w * _RPW + j * _CH, _CH)]

            def wstart(j, t):
                pltpu.async_copy(rows4.at[t], wslice(j), wsems[t])

            def wwait(j, t):
                pltpu.make_async_copy(rows4.at[t], wslice(j), wsems[t]).wait()

            for t in range(_NSLOT):
                gstart(t, t)

            def body(g, carry):
                for t in range(_NSLOT):
                    jprev = (g - 1) * _NSLOT + t
                    gwait(jprev, t)
                    wstart(jprev, t)
                for t in range(_NSLOT):
                    j = g * _NSLOT + t
                    wwait(j - _NSLOT, t)
                    gstart(j, t)
                return carry
            lax.fori_loop(1, _NGRP, body, 0)

            for t in range(_NSLOT):
                jlast = (_NGRP - 1) * _NSLOT + t
                gwait(jlast, t)
                wstart(jlast, t)
            for t in range(_NSLOT):
                jlast = (_NGRP - 1) * _NSLOT + t
                wwait(jlast, t)

        run_table(qf, idxq_v, qs)
        run_table(kvf, idxk_v, kvs)

    return k(qpad, kv, perm2d)


def _scatter_rows(rows_sorted, perm2d):
    @functools.partial(
        pl.kernel,
        out_type=jax.ShapeDtypeStruct((_ROWS, _DP), jnp.float32),
        scratch_types=[
            pltpu.VMEM((_NCH, _CH), jnp.int32),
            pltpu.VMEM((_NCH, _CH), jnp.float32),
            pltpu.VMEM((_NSLOT, _CH, _DP), jnp.float32),
        ] + [pltpu.SemaphoreType.DMA] * (2 * _NSLOT),
        mesh=_sc_mesh(),
        compiler_params=pltpu.CompilerParams(needs_layout_passes=False),
    )
    def k(src, perm, out, idx_v, idx_f, rows4, *sems):
        rsems, wsems = sems[:_NSLOT], sems[_NSLOT:]
        w = _wid()
        pltpu.sync_copy(perm.at[pl.ds(w * _NCH, _NCH)], idx_f)
        _bitcast_rows(idx_f, idx_v)

        def rslice(j):
            return src.at[pl.ds(w * _RPW + j * _CH, _CH)]

        def rstart(j, t):
            pltpu.async_copy(rslice(j), rows4.at[t], rsems[t])

        def rwait(j, t):
            pltpu.make_async_copy(rslice(j), rows4.at[t], rsems[t]).wait()

        def wstart(j, t):
            pltpu.async_copy(rows4.at[t], out.at[idx_v.at[j]], wsems[t])

        def wwait(j, t):
            pltpu.make_async_copy(
                rows4.at[t], out.at[idx_v.at[j]], wsems[t]).wait()

        for t in range(_NSLOT):
            rstart(t, t)

        def body(g, carry):
            for t in range(_NSLOT):
                jprev = (g - 1) * _NSLOT + t
                rwait(jprev, t)
                wstart(jprev, t)
            for t in range(_NSLOT):
                j = g * _NSLOT + t
                wwait(j - _NSLOT, t)
                rstart(j, t)
            return carry
        lax.fori_loop(1, _NGRP, body, 0)

        for t in range(_NSLOT):
            jlast = (_NGRP - 1) * _NSLOT + t
            rwait(jlast, t)
            wstart(jlast, t)
        for t in range(_NSLOT):
            jlast = (_NGRP - 1) * _NSLOT + t
            wwait(jlast, t)

    return k(rows_sorted, perm2d)


def _attn_body(qp_ref, kv_ref, samp_ref, out_ref, *, scale, n_over_m):
    for hh in range(_HB):
        qb = qp_ref[hh][:, :_D]
        kb = kv_ref[hh][:, :_D]
        vb = kv_ref[hh][:, _D:]
        ks = samp_ref[hh][:, :_D]
        vs = samp_ref[hh][:, _D:]

        s1 = jax.lax.dot_general(qb, kb, (((1,), (1,)), ((), ())),
                                 preferred_element_type=jnp.float32) * scale
        p1 = jnp.exp(s1)
        l1 = jnp.sum(p1, axis=-1)
        o1 = jax.lax.dot_general(p1, vb, (((1,), (0,)), ((), ())),
                                 preferred_element_type=jnp.float32)

        s2 = jax.lax.dot_general(qb, ks, (((1,), (1,)), ((), ())),
                                 preferred_element_type=jnp.float32) * scale
        p2 = jnp.exp(s2)
        l2 = jnp.sum(p2, axis=-1)
        o2 = jax.lax.dot_general(p2, vs, (((1,), (0,)), ((), ())),
                                 preferred_element_type=jnp.float32)

        den = l1 + n_over_m * l2
        out_ref[hh, :, :_D] = (o1 + n_over_m * o2) / den[:, None]
        out_ref[hh, :, _D:] = jnp.zeros((qb.shape[0], _DP - _D), jnp.float32)


def _fused_attention(qs_pad, kvs, samp):
    bs = BLOCK_SIZE
    nb = _S // bs
    m = samp.shape[1]
    scale = 1.0 / (_D ** 0.5)
    n_over_m = float(_S) / float(m)

    body = functools.partial(_attn_body, scale=scale, n_over_m=n_over_m)
    return pl.pallas_call(
        body,
        grid=(_H // _HB, nb),
        in_specs=[
            pl.BlockSpec((_HB, bs, _DP), lambda h, i: (h, i, 0)),
            pl.BlockSpec((_HB, bs, _DP), lambda h, i: (h, i, 0)),
            pl.BlockSpec((_HB, m, _DP), lambda h, i: (h, 0, 0)),
        ],
        out_specs=pl.BlockSpec((_HB, bs, _DP), lambda h, i: (h, i, 0)),
        out_shape=jax.ShapeDtypeStruct((_H, _S, _DP), jnp.float32),
    )(qs_pad, kvs, samp)


def kernel(q, k, v, R):
    B, H, S, D = q.shape
    assert (B, H, S, D) == (1, _H, _S, _D)

    q0, k0, v0 = q[0], k[0], v[0]
    qh2d, kh2d = _hash(q0, k0, R)
    perm2d = _sort_codes(qh2d, kh2d)
    qpad, kv = _pack(q0, k0, v0)

    qsf, kvsf = _gather_rows(qpad, kv, perm2d)
    qs_pad = qsf.reshape(_H, _S, _DP)
    kvs = kvsf.reshape(_H, _S, _DP)

    stride = _S // SAMPLE_SIZE
    samp = kv.reshape(_H, _S, _DP)[:, ::stride, :]

    out_sorted = _fused_attention(qs_pad, kvs, samp)

    outf = _scatter_rows(out_sorted.reshape(_ROWS, _DP), perm2d)
    return outf[:, :_D].reshape(1, _H, _S, _D)


# speedup vs baseline: 1.0264x; 1.0264x over previous
import functools

import jax
import jax.numpy as jnp
from jax import lax
from jax.experimental import pallas as pl
from jax.experimental.pallas import tpu as pltpu
from jax.experimental.pallas import tpu_sc as plsc

NUM_HASH = 16
BLOCK_SIZE = 256
SAMPLE_SIZE = 256
_HB = 2

_NC = 2
_NS = 16
_NW = _NC * _NS

_H = 12
_S = 8192
_D = 64
_DP = 128
_ROWS = _H * _S
_RPW = _ROWS // _NW
_CH = 128
_NCH = _RPW // _CH


def _sc_mesh():
    return plsc.VectorSubcoreMesh(core_axis_name="c", subcore_axis_name="s")


def _wid():
    return lax.axis_index("s") * _NC + lax.axis_index("c")


def _hash_body(q_ref, k_ref, r_ref, qh_ref, kh_ref):
    R = r_ref[...]
    w = 2 ** lax.broadcasted_iota(jnp.int32, (1, NUM_HASH), 1)

    pq = jax.lax.dot_general(q_ref[0], R, (((1,), (0,)), ((), ())),
                             preferred_element_type=jnp.float32)
    pk = jax.lax.dot_general(k_ref[0], R, (((1,), (0,)), ((), ())),
                             preferred_element_type=jnp.float32)
    qh_ref[...] = jnp.sum(jnp.where(pq > 0, w, 0), axis=-1).astype(
        jnp.float32).reshape(_S // _CH, _CH)
    kh_ref[...] = jnp.sum(jnp.where(pk > 0, w, 0), axis=-1).astype(
        jnp.float32).reshape(_S // _CH, _CH)


def _hash(q0, k0, R):
    out_types = (
        jax.ShapeDtypeStruct((_ROWS // _CH, _CH), jnp.float32),
        jax.ShapeDtypeStruct((_ROWS // _CH, _CH), jnp.float32),
    )
    return pl.pallas_call(
        _hash_body,
        grid=(_H,),
        in_specs=[
            pl.BlockSpec((1, _S, _D), lambda h: (h, 0, 0)),
            pl.BlockSpec((1, _S, _D), lambda h: (h, 0, 0)),
            pl.BlockSpec((_D, NUM_HASH), lambda h: (0, 0)),
        ],
        out_specs=[
            pl.BlockSpec((_S // _CH, _CH), lambda h: (h, 0)),
            pl.BlockSpec((_S // _CH, _CH), lambda h: (h, 0)),
        ],
        out_shape=out_types,
    )(q0, k0, R)


def _pack_body(q_ref, k_ref, v_ref, qpad_ref, kv_ref):
    qb = q_ref[0]
    qpad_ref[0, :, :_D] = qb
    qpad_ref[0, :, _D:] = jnp.zeros_like(qb)
    kv_ref[0, :, :_D] = k_ref[0]
    kv_ref[0, :, _D:] = v_ref[0]


def _pack(q0, k0, v0):
    out_types = (
        jax.ShapeDtypeStruct((_H, _S, _DP), jnp.float32),
        jax.ShapeDtypeStruct((_H, _S, _DP), jnp.float32),
    )
    qpad, kv = pl.pallas_call(
        _pack_body,
        grid=(_H,),
        in_specs=[
            pl.BlockSpec((1, _S, _D), lambda h: (h, 0, 0)),
            pl.BlockSpec((1, _S, _D), lambda h: (h, 0, 0)),
            pl.BlockSpec((1, _S, _D), lambda h: (h, 0, 0)),
        ],
        out_specs=[
            pl.BlockSpec((1, _S, _DP), lambda h: (h, 0, 0)),
            pl.BlockSpec((1, _S, _DP), lambda h: (h, 0, 0)),
        ],
        out_shape=out_types,
    )(q0, k0, v0)
    return qpad.reshape(_ROWS, _DP), kv.reshape(_ROWS, _DP)


_NCODES = 1 << NUM_HASH
_NSORT = 2 * _H


_L = 16
_CPL = _S // _L


def _lg(ref, addr):
    if len(ref.shape) == 2:
        v = plsc.load_gather(ref, [addr >> 7, addr & 127])
        return v.astype(jnp.int32) if v.dtype == jnp.float32 else v
    return plsc.load_gather(ref, [addr])


def _ss(ref, addr, val):
    if ref.dtype == jnp.float32 and val.dtype == jnp.int32:
        val = plsc.bitcast(val, jnp.float32)
    if len(ref.shape) == 2:
        plsc.store_scatter(ref, [addr >> 7, addr & 127], val)
    else:
        plsc.store_scatter(ref, [addr], val)


def _sort_codes(qh, kh):
    @functools.partial(
        pl.kernel,
        out_type=jax.ShapeDtypeStruct((2 * _ROWS // _CH, _CH), jnp.float32),
        scratch_types=[
            pltpu.VMEM((_S // _CH, _CH), jnp.float32),
            pltpu.VMEM((_S,), jnp.int32),
            pltpu.VMEM((_S,), jnp.int32),
            pltpu.VMEM((_S // _CH, _CH), jnp.float32),
            pltpu.VMEM((256 * _L,), jnp.int32),
        ],
        mesh=_sc_mesh(),
        compiler_params=pltpu.CompilerParams(needs_layout_passes=False),
    )
    def k(qh_hbm, kh_hbm, perm_hbm, c0, c1, v1, v2, hist):
        w = _wid()

        @pl.when(w < _H)
        def _():
            pltpu.sync_copy(qh_hbm.at[pl.ds(w * (_S // _CH), _S // _CH)], c0)

        @pl.when((w >= _H) & (w < _NSORT))
        def _():
            pltpu.sync_copy(
                kh_hbm.at[pl.ds((w - _H) * (_S // _CH), _S // _CH)], c0)

        @pl.when(w < _NSORT)
        def _():
            lane = jax.lax.iota(jnp.int32, 16)
            zeros = jnp.zeros((16,), jnp.int32)

            def radix_pass(src_c, src_v, dst_c, dst_v, shift, base):
                def zb(j, c):
                    hist[pl.ds(j * 16, 16)] = zeros
                    return c
                lax.fori_loop(0, 256, zb, 0)

                def hb(j, c):
                    addr = lane * _CPL + j
                    cv = _lg(src_c, addr)
                    digit = (cv >> shift) & 255
                    haddr = digit * _L + lane
                    cur = plsc.load_gather(hist, [haddr])
                    plsc.store_scatter(hist, [haddr], cur + 1)
                    return c
                lax.fori_loop(0, _CPL, hb, 0)

                def sb(j, carry):
                    vec = hist[pl.ds(j * 16, 16)]
                    total = jnp.sum(vec)
                    hist[pl.ds(j * 16, 16)] = plsc.cumsum(vec) - vec + carry
                    return carry + total
                lax.fori_loop(0, 256, sb, jnp.int32(0))

                def pb(j, c):
                    addr = lane * _CPL + j
                    cv = _lg(src_c, addr)
                    vv = addr if src_v is None else plsc.load_gather(
                        src_v, [addr])
                    digit = (cv >> shift) & 255
                    haddr = digit * _L + lane
                    pos = plsc.load_gather(hist, [haddr])
                    plsc.store_scatter(hist, [haddr], pos + 1)
                    if dst_c is not None:
                        plsc.store_scatter(dst_c, [pos], cv)
                    _ss(dst_v, pos, vv + base)
                    return c
                lax.fori_loop(0, _CPL, pb, 0)

            radix_pass(c0, None, c1, v1, 0, 0)
            radix_pass(c1, v1, None, v2, 8, (w % _H) * _S)
            row0 = jnp.where(w < _H, w * (_S // _CH),
                             _ROWS // _CH + (w - _H) * (_S // _CH))
            pltpu.sync_copy(v2, perm_hbm.at[pl.ds(row0, _S // _CH)])

    return k(qh, kh)


_NSLOT = 4
_NGRP = _NCH // _NSLOT


def _bitcast_rows(src_f, dst_i, nch):
    lane = jax.lax.iota(jnp.int32, 16)

    def body(t, c):
        a = t * 16 + lane
        v = plsc.load_gather(src_f, [a >> 7, a & 127])
        plsc.store_scatter(dst_i, [a >> 7, a & 127],
                           plsc.bitcast(v, jnp.int32))
        return c
    lax.fori_loop(0, nch * _CH // 16, body, 0)


_HHALF = _H // 2
_ROWS_HF = _ROWS // 2
_RPW_HF = _ROWS_HF // _NW
_NCH_HF = _RPW_HF // _CH
_PR_HF = _ROWS_HF // _CH


def _ring(gstart, gwait, wstart, wwait, nch):
    ngrp = nch // _NSLOT
    for t in range(_NSLOT):
        gstart(t, t)

    def body(g, carry):
        for t in range(_NSLOT):
            jprev = (g - 1) * _NSLOT + t
            gwait(jprev, t)
            wstart(jprev, t)
        for t in range(_NSLOT):
            j = g * _NSLOT + t
            wwait(j - _NSLOT, t)
            gstart(j, t)
        return carry
    lax.fori_loop(1, ngrp, body, 0)

    for t in range(_NSLOT):
        jlast = (ngrp - 1) * _NSLOT + t
        gwait(jlast, t)
        wstart(jlast, t)
    for t in range(_NSLOT):
        jlast = (ngrp - 1) * _NSLOT + t
        wwait(jlast, t)


def _gather_rows(qpad, kv, perm2d, half):
    out_t = jax.ShapeDtypeStruct((_ROWS_HF, _DP), jnp.float32)

    @functools.partial(
        pl.kernel,
        out_type=(out_t, out_t),
        scratch_types=[
            pltpu.VMEM((_NCH, _CH), jnp.int32),
            pltpu.VMEM((_NCH, _CH), jnp.float32),
            pltpu.VMEM((_NSLOT, _CH, _DP), jnp.float32),
        ] + [pltpu.SemaphoreType.DMA] * (2 * _NSLOT),
        mesh=_sc_mesh(),
        compiler_params=pltpu.CompilerParams(needs_layout_passes=False),
    )
    def k(qf, kvf, perm, qs, kvs, idx_v, idx_f, rows4, *sems):
        gsems, wsems = sems[:_NSLOT], sems[_NSLOT:]
        w = _wid()
        nwh = _NW // 2

        def run_table(tab, wloc, perm_base, out):
            pltpu.sync_copy(
                perm.at[pl.ds(perm_base + wloc * _NCH, _NCH)], idx_f)
            _bitcast_rows(idx_f, idx_v, _NCH)

            def gstart(j, t):
                pltpu.async_copy(tab.at[idx_v.at[j]], rows4.at[t], gsems[t])

            def gwait(j, t):
                pltpu.make_async_copy(
                    tab.at[idx_v.at[j]], rows4.at[t], gsems[t]).wait()

            def wslice(j):
                return out.at[pl.ds(wloc * _RPW + j * _CH, _CH)]

            def wstart(j, t):
                pltpu.async_copy(rows4.at[t], wslice(j), wsems[t])

            def wwait(j, t):
                pltpu.make_async_copy(rows4.at[t], wslice(j), wsems[t]).wait()

            _ring(gstart, gwait, wstart, wwait, _NCH)

        @pl.when(w < nwh)
        def _():
            run_table(qf, w, half * _PR_HF, qs)

        @pl.when(w >= nwh)
        def _():
            run_table(kvf, w - nwh, _ROWS // _CH + half * _PR_HF, kvs)

    return k(qpad, kv, perm2d)


def _scatter_rows(rows_sorted, perm2d):
    @functools.partial(
        pl.kernel,
        out_type=jax.ShapeDtypeStruct((_ROWS, _DP), jnp.float32),
        scratch_types=[
            pltpu.VMEM((_NCH, _CH), jnp.int32),
            pltpu.VMEM((_NCH, _CH), jnp.float32),
            pltpu.VMEM((_NSLOT, _CH, _DP), jnp.float32),
        ] + [pltpu.SemaphoreType.DMA] * (2 * _NSLOT),
        mesh=_sc_mesh(),
        compiler_params=pltpu.CompilerParams(needs_layout_passes=False),
    )
    def k(src0, src1, perm, out, idx_v, idx_f, rows4, *sems):
        rsems, wsems = sems[:_NSLOT], sems[_NSLOT:]
        w = _wid()
        pltpu.sync_copy(perm.at[pl.ds(w * _NCH, _NCH)], idx_f)
        _bitcast_rows(idx_f, idx_v, _NCH)

        def run_half(src, wloc):
            def rslice(j):
                return src.at[pl.ds(wloc * _RPW + j * _CH, _CH)]

            def rstart(j, t):
                pltpu.async_copy(rslice(j), rows4.at[t], rsems[t])

            def rwait(j, t):
                pltpu.make_async_copy(rslice(j), rows4.at[t], rsems[t]).wait()

            def wstart(j, t):
                pltpu.async_copy(rows4.at[t], out.at[idx_v.at[j]], wsems[t])

            def wwait(j, t):
                pltpu.make_async_copy(
                    rows4.at[t], out.at[idx_v.at[j]], wsems[t]).wait()

            _ring(rstart, rwait, wstart, wwait, _NCH)

        @pl.when(w < _NW // 2)
        def _():
            run_half(src0, w)

        @pl.when(w >= _NW // 2)
        def _():
            run_half(src1, w - _NW // 2)

    return k(rows_sorted[0], rows_sorted[1], perm2d)


def _attn_body(qp_ref, kv_ref, samp_ref, out_ref, *, scale, n_over_m):
    for hh in range(_HB):
        qb = qp_ref[hh][:, :_D]
        kb = kv_ref[hh][:, :_D]
        vb = kv_ref[hh][:, _D:]
        ks = samp_ref[hh][:, :_D]
        vs = samp_ref[hh][:, _D:]

        s1 = jax.lax.dot_general(qb, kb, (((1,), (1,)), ((), ())),
                                 preferred_element_type=jnp.float32) * scale
        p1 = jnp.exp(s1)
        l1 = jnp.sum(p1, axis=-1)
        o1 = jax.lax.dot_general(p1, vb, (((1,), (0,)), ((), ())),
                                 preferred_element_type=jnp.float32)

        s2 = jax.lax.dot_general(qb, ks, (((1,), (1,)), ((), ())),
                                 preferred_element_type=jnp.float32) * scale
        p2 = jnp.exp(s2)
        l2 = jnp.sum(p2, axis=-1)
        o2 = jax.lax.dot_general(p2, vs, (((1,), (0,)), ((), ())),
                                 preferred_element_type=jnp.float32)

        den = l1 + n_over_m * l2
        out_ref[hh, :, :_D] = (o1 + n_over_m * o2) / den[:, None]
        out_ref[hh, :, _D:] = jnp.zeros((qb.shape[0], _DP - _D), jnp.float32)


def _fused_attention(qs_pad, kvs, samp):
    bs = BLOCK_SIZE
    nb = _S // bs
    nh = qs_pad.shape[0]
    m = samp.shape[1]
    scale = 1.0 / (_D ** 0.5)
    n_over_m = float(_S) / float(m)

    body = functools.partial(_attn_body, scale=scale, n_over_m=n_over_m)
    return pl.pallas_call(
        body,
        grid=(nh // _HB, nb),
        in_specs=[
            pl.BlockSpec((_HB, bs, _DP), lambda h, i: (h, i, 0)),
            pl.BlockSpec((_HB, bs, _DP), lambda h, i: (h, i, 0)),
            pl.BlockSpec((_HB, m, _DP), lambda h, i: (h, 0, 0)),
        ],
        out_specs=pl.BlockSpec((_HB, bs, _DP), lambda h, i: (h, i, 0)),
        out_shape=jax.ShapeDtypeStruct((nh, _S, _DP), jnp.float32),
    )(qs_pad, kvs, samp)


def kernel(q, k, v, R):
    B, H, S, D = q.shape
    assert (B, H, S, D) == (1, _H, _S, _D)

    q0, k0, v0 = q[0], k[0], v[0]
    qh2d, kh2d = _hash(q0, k0, R)
    perm2d = _sort_codes(qh2d, kh2d)
    qpad, kv = _pack(q0, k0, v0)

    stride = _S // SAMPLE_SIZE
    samp = kv.reshape(_H, _S, _DP)[:, ::stride, :]

    halves = []
    gathered = [_gather_rows(qpad, kv, perm2d, hf) for hf in (0, 1)]
    for hf, (qsf, kvsf) in enumerate(gathered):
        qs_pad = qsf.reshape(_HHALF, _S, _DP)
        kvs = kvsf.reshape(_HHALF, _S, _DP)
        s = samp[hf * _HHALF:(hf + 1) * _HHALF]
        halves.append(_fused_attention(qs_pad, kvs, s).reshape(
            _ROWS_HF, _DP))

    outf = _scatter_rows((halves[0], halves[1]), perm2d)
    return outf[:, :_D].reshape(1, _H, _S, _D)


# speedup vs baseline: 1.0940x; 1.0658x over previous
import functools

import jax
import jax.numpy as jnp
from jax import lax
from jax.experimental import pallas as pl
from jax.experimental.pallas import tpu as pltpu
from jax.experimental.pallas import tpu_sc as plsc

NUM_HASH = 16
BLOCK_SIZE = 256
SAMPLE_SIZE = 256
_HB = 3

_NC = 2
_NS = 16
_NW = _NC * _NS

_H = 12
_S = 8192
_D = 64
_DP = 128
_ROWS = _H * _S
_RPW = _ROWS // _NW
_CH = 128
_NCH = _RPW // _CH


def _sc_mesh():
    return plsc.VectorSubcoreMesh(core_axis_name="c", subcore_axis_name="s")


def _wid():
    return lax.axis_index("s") * _NC + lax.axis_index("c")


def _hash_body(q_ref, k_ref, r_ref, qh_ref, kh_ref):
    R = r_ref[...]
    w = 2 ** lax.broadcasted_iota(jnp.int32, (1, NUM_HASH), 1)

    pq = jax.lax.dot_general(q_ref[0], R, (((1,), (0,)), ((), ())),
                             preferred_element_type=jnp.float32)
    pk = jax.lax.dot_general(k_ref[0], R, (((1,), (0,)), ((), ())),
                             preferred_element_type=jnp.float32)
    qh_ref[...] = jnp.sum(jnp.where(pq > 0, w, 0), axis=-1).astype(
        jnp.float32).reshape(_S // _CH, _CH)
    kh_ref[...] = jnp.sum(jnp.where(pk > 0, w, 0), axis=-1).astype(
        jnp.float32).reshape(_S // _CH, _CH)


def _hash(q0, k0, R):
    out_types = (
        jax.ShapeDtypeStruct((_ROWS // _CH, _CH), jnp.float32),
        jax.ShapeDtypeStruct((_ROWS // _CH, _CH), jnp.float32),
    )
    return pl.pallas_call(
        _hash_body,
        grid=(_H,),
        in_specs=[
            pl.BlockSpec((1, _S, _D), lambda h: (h, 0, 0)),
            pl.BlockSpec((1, _S, _D), lambda h: (h, 0, 0)),
            pl.BlockSpec((_D, NUM_HASH), lambda h: (0, 0)),
        ],
        out_specs=[
            pl.BlockSpec((_S // _CH, _CH), lambda h: (h, 0)),
            pl.BlockSpec((_S // _CH, _CH), lambda h: (h, 0)),
        ],
        out_shape=out_types,
    )(q0, k0, R)


def _pack_body(q_ref, k_ref, v_ref, qpad_ref, kv_ref):
    qb = q_ref[0]
    qpad_ref[0, :, :_D] = qb
    qpad_ref[0, :, _D:] = jnp.zeros_like(qb)
    kv_ref[0, :, :_D] = k_ref[0]
    kv_ref[0, :, _D:] = v_ref[0]


def _pack(q0, k0, v0):
    out_types = (
        jax.ShapeDtypeStruct((_H, _S, _DP), jnp.float32),
        jax.ShapeDtypeStruct((_H, _S, _DP), jnp.float32),
    )
    qpad, kv = pl.pallas_call(
        _pack_body,
        grid=(_H,),
        in_specs=[
            pl.BlockSpec((1, _S, _D), lambda h: (h, 0, 0)),
            pl.BlockSpec((1, _S, _D), lambda h: (h, 0, 0)),
            pl.BlockSpec((1, _S, _D), lambda h: (h, 0, 0)),
        ],
        out_specs=[
            pl.BlockSpec((1, _S, _DP), lambda h: (h, 0, 0)),
            pl.BlockSpec((1, _S, _DP), lambda h: (h, 0, 0)),
        ],
        out_shape=out_types,
    )(q0, k0, v0)
    return qpad.reshape(_ROWS, _DP), kv.reshape(_ROWS, _DP)


_NCODES = 1 << NUM_HASH
_NSORT = 2 * _H


_L = 16
_CPL = _S // _L


def _lg(ref, addr):
    if len(ref.shape) == 2:
        v = plsc.load_gather(ref, [addr >> 7, addr & 127])
        return v.astype(jnp.int32) if v.dtype == jnp.float32 else v
    return plsc.load_gather(ref, [addr])


def _ss(ref, addr, val):
    if ref.dtype == jnp.float32 and val.dtype == jnp.int32:
        val = plsc.bitcast(val, jnp.float32)
    if len(ref.shape) == 2:
        plsc.store_scatter(ref, [addr >> 7, addr & 127], val)
    else:
        plsc.store_scatter(ref, [addr], val)


def _sort_codes(qh, kh):
    @functools.partial(
        pl.kernel,
        out_type=jax.ShapeDtypeStruct((2 * _ROWS // _CH, _CH), jnp.float32),
        scratch_types=[
            pltpu.VMEM((_S // _CH, _CH), jnp.float32),
            pltpu.VMEM((_S,), jnp.int32),
            pltpu.VMEM((_S,), jnp.int32),
            pltpu.VMEM((_S // _CH, _CH), jnp.float32),
            pltpu.VMEM((256 * _L,), jnp.int32),
        ],
        mesh=_sc_mesh(),
        compiler_params=pltpu.CompilerParams(needs_layout_passes=False),
    )
    def k(qh_hbm, kh_hbm, perm_hbm, c0, c1, v1, v2, hist):
        w = _wid()

        @pl.when(w < _H)
        def _():
            pltpu.sync_copy(qh_hbm.at[pl.ds(w * (_S // _CH), _S // _CH)], c0)

        @pl.when((w >= _H) & (w < _NSORT))
        def _():
            pltpu.sync_copy(
                kh_hbm.at[pl.ds((w - _H) * (_S // _CH), _S // _CH)], c0)

        @pl.when(w < _NSORT)
        def _():
            lane = jax.lax.iota(jnp.int32, 16)
            zeros = jnp.zeros((16,), jnp.int32)

            def radix_pass(src_c, src_v, dst_c, dst_v, shift, base):
                def zb(j, c):
                    hist[pl.ds(j * 16, 16)] = zeros
                    return c
                lax.fori_loop(0, 256, zb, 0)

                def hb(j, c):
                    addr = lane * _CPL + j
                    cv = _lg(src_c, addr)
                    digit = (cv >> shift) & 255
                    haddr = digit * _L + lane
                    cur = plsc.load_gather(hist, [haddr])
                    plsc.store_scatter(hist, [haddr], cur + 1)
                    return c
                lax.fori_loop(0, _CPL, hb, 0)

                def sb(j, carry):
                    vec = hist[pl.ds(j * 16, 16)]
                    total = jnp.sum(vec)
                    hist[pl.ds(j * 16, 16)] = plsc.cumsum(vec) - vec + carry
                    return carry + total
                lax.fori_loop(0, 256, sb, jnp.int32(0))

                def pb(j, c):
                    addr = lane * _CPL + j
                    cv = _lg(src_c, addr)
                    vv = addr if src_v is None else plsc.load_gather(
                        src_v, [addr])
                    digit = (cv >> shift) & 255
                    haddr = digit * _L + lane
                    pos = plsc.load_gather(hist, [haddr])
                    plsc.store_scatter(hist, [haddr], pos + 1)
                    if dst_c is not None:
                        plsc.store_scatter(dst_c, [pos], cv)
                    _ss(dst_v, pos, vv + base)
                    return c
                lax.fori_loop(0, _CPL, pb, 0)

            radix_pass(c0, None, c1, v1, 0, 0)
            radix_pass(c1, v1, None, v2, 8, (w % _H) * _S)
            row0 = jnp.where(w < _H, w * (_S // _CH),
                             _ROWS // _CH + (w - _H) * (_S // _CH))
            pltpu.sync_copy(v2, perm_hbm.at[pl.ds(row0, _S // _CH)])

    return k(qh, kh)


_NSLOT = 4
_NGRP = _NCH // _NSLOT


def _bitcast_rows(src_f, dst_i, nch):
    lane = jax.lax.iota(jnp.int32, 16)

    def body(t, c):
        a = t * 16 + lane
        v = plsc.load_gather(src_f, [a >> 7, a & 127])
        plsc.store_scatter(dst_i, [a >> 7, a & 127],
                           plsc.bitcast(v, jnp.int32))
        return c
    lax.fori_loop(0, nch * _CH // 16, body, 0)


_HHALF = _H // 2
_ROWS_HF = _ROWS // 2
_RPW_HF = _ROWS_HF // _NW
_NCH_HF = _RPW_HF // _CH
_PR_HF = _ROWS_HF // _CH


def _ring(gstart, gwait, wstart, wwait, nch):
    ngrp = nch // _NSLOT
    for t in range(_NSLOT):
        gstart(t, t)

    def body(g, carry):
        for t in range(_NSLOT):
            jprev = (g - 1) * _NSLOT + t
            gwait(jprev, t)
            wstart(jprev, t)
        for t in range(_NSLOT):
            j = g * _NSLOT + t
            wwait(j - _NSLOT, t)
            gstart(j, t)
        return carry
    lax.fori_loop(1, ngrp, body, 0)

    for t in range(_NSLOT):
        jlast = (ngrp - 1) * _NSLOT + t
        gwait(jlast, t)
        wstart(jlast, t)
    for t in range(_NSLOT):
        jlast = (ngrp - 1) * _NSLOT + t
        wwait(jlast, t)


def _gather_rows(qpad, kv, perm2d, half):
    out_t = jax.ShapeDtypeStruct((_ROWS_HF, _DP), jnp.float32)

    @functools.partial(
        pl.kernel,
        out_type=(out_t, out_t),
        scratch_types=[
            pltpu.VMEM((_NCH, _CH), jnp.int32),
            pltpu.VMEM((_NCH, _CH), jnp.float32),
            pltpu.VMEM((_NSLOT, _CH, _DP), jnp.float32),
        ] + [pltpu.SemaphoreType.DMA] * (2 * _NSLOT),
        mesh=_sc_mesh(),
        compiler_params=pltpu.CompilerParams(needs_layout_passes=False),
    )
    def k(qf, kvf, perm, qs, kvs, idx_v, idx_f, rows4, *sems):
        gsems, wsems = sems[:_NSLOT], sems[_NSLOT:]
        w = _wid()
        nwh = _NW // 2

        def run_table(tab, wloc, perm_base, out):
            pltpu.sync_copy(
                perm.at[pl.ds(perm_base + wloc * _NCH, _NCH)], idx_f)
            _bitcast_rows(idx_f, idx_v, _NCH)

            def gstart(j, t):
                pltpu.async_copy(tab.at[idx_v.at[j]], rows4.at[t], gsems[t])

            def gwait(j, t):
                pltpu.make_async_copy(
                    tab.at[idx_v.at[j]], rows4.at[t], gsems[t]).wait()

            def wslice(j):
                return out.at[pl.ds(wloc * _RPW + j * _CH, _CH)]

            def wstart(j, t):
                pltpu.async_copy(rows4.at[t], wslice(j), wsems[t])

            def wwait(j, t):
                pltpu.make_async_copy(rows4.at[t], wslice(j), wsems[t]).wait()

            _ring(gstart, gwait, wstart, wwait, _NCH)

        @pl.when(w < nwh)
        def _():
            run_table(qf, w, half * _PR_HF, qs)

        @pl.when(w >= nwh)
        def _():
            run_table(kvf, w - nwh, _ROWS // _CH + half * _PR_HF, kvs)

    return k(qpad, kv, perm2d)


def _scatter_rows(rows_sorted, perm2d):
    @functools.partial(
        pl.kernel,
        out_type=jax.ShapeDtypeStruct((_ROWS, _DP), jnp.float32),
        scratch_types=[
            pltpu.VMEM((_NCH, _CH), jnp.int32),
            pltpu.VMEM((_NCH, _CH), jnp.float32),
            pltpu.VMEM((_NSLOT, _CH, _DP), jnp.float32),
        ] + [pltpu.SemaphoreType.DMA] * (2 * _NSLOT),
        mesh=_sc_mesh(),
        compiler_params=pltpu.CompilerParams(needs_layout_passes=False),
    )
    def k(src0, src1, perm, out, idx_v, idx_f, rows4, *sems):
        rsems, wsems = sems[:_NSLOT], sems[_NSLOT:]
        w = _wid()
        pltpu.sync_copy(perm.at[pl.ds(w * _NCH, _NCH)], idx_f)
        _bitcast_rows(idx_f, idx_v, _NCH)

        def run_half(src, wloc):
            def rslice(j):
                return src.at[pl.ds(wloc * _RPW + j * _CH, _CH)]

            def rstart(j, t):
                pltpu.async_copy(rslice(j), rows4.at[t], rsems[t])

            def rwait(j, t):
                pltpu.make_async_copy(rslice(j), rows4.at[t], rsems[t]).wait()

            def wstart(j, t):
                pltpu.async_copy(rows4.at[t], out.at[idx_v.at[j]], wsems[t])

            def wwait(j, t):
                pltpu.make_async_copy(
                    rows4.at[t], out.at[idx_v.at[j]], wsems[t]).wait()

            _ring(rstart, rwait, wstart, wwait, _NCH)

        @pl.when(w < _NW // 2)
        def _():
            run_half(src0, w)

        @pl.when(w >= _NW // 2)
        def _():
            run_half(src1, w - _NW // 2)

    return k(rows_sorted[0], rows_sorted[1], perm2d)


def _attn_body(qp_ref, kv_ref, samp_ref, out_ref, *, scale, n_over_m):
    for hh in range(_HB):
        qb = qp_ref[hh][:, :_D]
        kb = kv_ref[hh][:, :_D]
        vb = kv_ref[hh][:, _D:]
        ks = samp_ref[hh][:, :_D]
        vs = samp_ref[hh][:, _D:]

        s1 = jax.lax.dot_general(qb, kb, (((1,), (1,)), ((), ())),
                                 preferred_element_type=jnp.float32) * scale
        p1 = jnp.exp(s1)
        l1 = jnp.sum(p1, axis=-1)
        o1 = jax.lax.dot_general(p1, vb, (((1,), (0,)), ((), ())),
                                 preferred_element_type=jnp.float32)

        s2 = jax.lax.dot_general(qb, ks, (((1,), (1,)), ((), ())),
                                 preferred_element_type=jnp.float32) * scale
        p2 = jnp.exp(s2)
        l2 = jnp.sum(p2, axis=-1)
        o2 = jax.lax.dot_general(p2, vs, (((1,), (0,)), ((), ())),
                                 preferred_element_type=jnp.float32)

        den = l1 + n_over_m * l2
        out_ref[hh, :, :_D] = (o1 + n_over_m * o2) / den[:, None]
        out_ref[hh, :, _D:] = jnp.zeros((qb.shape[0], _DP - _D), jnp.float32)


def _fused_attention(qs_pad, kvs, samp):
    bs = BLOCK_SIZE
    nb = _S // bs
    nh = qs_pad.shape[0]
    m = samp.shape[1]
    scale = 1.0 / (_D ** 0.5)
    n_over_m = float(_S) / float(m)

    body = functools.partial(_attn_body, scale=scale, n_over_m=n_over_m)
    return pl.pallas_call(
        body,
        grid=(nh // _HB, nb),
        in_specs=[
            pl.BlockSpec((_HB, bs, _DP), lambda h, i: (h, i, 0)),
            pl.BlockSpec((_HB, bs, _DP), lambda h, i: (h, i, 0)),
            pl.BlockSpec((_HB, m, _DP), lambda h, i: (h, 0, 0)),
        ],
        out_specs=pl.BlockSpec((_HB, bs, _DP), lambda h, i: (h, i, 0)),
        out_shape=jax.ShapeDtypeStruct((nh, _S, _DP), jnp.float32),
    )(qs_pad, kvs, samp)


def kernel(q, k, v, R):
    B, H, S, D = q.shape
    assert (B, H, S, D) == (1, _H, _S, _D)

    q0, k0, v0 = q[0], k[0], v[0]
    qh2d, kh2d = _hash(q0, k0, R)
    perm2d = _sort_codes(qh2d, kh2d)
    qpad, kv = _pack(q0, k0, v0)

    stride = _S // SAMPLE_SIZE
    samp = kv.reshape(_H, _S, _DP)[:, ::stride, :]

    halves = []
    gathered = [_gather_rows(qpad, kv, perm2d, hf) for hf in (0, 1)]
    for hf, (qsf, kvsf) in enumerate(gathered):
        qs_pad = qsf.reshape(_HHALF, _S, _DP)
        kvs = kvsf.reshape(_HHALF, _S, _DP)
        s = samp[hf * _HHALF:(hf + 1) * _HHALF]
        halves.append(_fused_attention(qs_pad, kvs, s).reshape(
            _ROWS_HF, _DP))

    outf = _scatter_rows((halves[0], halves[1]), perm2d)
    return outf[:, :_D].reshape(1, _H, _S, _D)


# speedup vs baseline: 1.1703x; 1.0698x over previous
import functools

import jax
import jax.numpy as jnp
from jax import lax
from jax.experimental import pallas as pl
from jax.experimental.pallas import tpu as pltpu
from jax.experimental.pallas import tpu_sc as plsc

NUM_HASH = 16
BLOCK_SIZE = 256
SAMPLE_SIZE = 256
_HB = 6

_NC = 2
_NS = 16
_NW = _NC * _NS

_H = 12
_S = 8192
_D = 64
_DP = 128
_ROWS = _H * _S
_RPW = _ROWS // _NW
_CH = 128
_NCH = _RPW // _CH


def _sc_mesh():
    return plsc.VectorSubcoreMesh(core_axis_name="c", subcore_axis_name="s")


def _wid():
    return lax.axis_index("s") * _NC + lax.axis_index("c")


def _hash_body(q_ref, k_ref, r_ref, qh_ref, kh_ref):
    R = r_ref[...]
    w = 2 ** lax.broadcasted_iota(jnp.int32, (1, NUM_HASH), 1)

    pq = jax.lax.dot_general(q_ref[0], R, (((1,), (0,)), ((), ())),
                             preferred_element_type=jnp.float32)
    pk = jax.lax.dot_general(k_ref[0], R, (((1,), (0,)), ((), ())),
                             preferred_element_type=jnp.float32)
    qh_ref[...] = jnp.sum(jnp.where(pq > 0, w, 0), axis=-1).astype(
        jnp.float32).reshape(_S // _CH, _CH)
    kh_ref[...] = jnp.sum(jnp.where(pk > 0, w, 0), axis=-1).astype(
        jnp.float32).reshape(_S // _CH, _CH)


def _hash(q0, k0, R):
    out_types = (
        jax.ShapeDtypeStruct((_ROWS // _CH, _CH), jnp.float32),
        jax.ShapeDtypeStruct((_ROWS // _CH, _CH), jnp.float32),
    )
    return pl.pallas_call(
        _hash_body,
        grid=(_H,),
        in_specs=[
            pl.BlockSpec((1, _S, _D), lambda h: (h, 0, 0)),
            pl.BlockSpec((1, _S, _D), lambda h: (h, 0, 0)),
            pl.BlockSpec((_D, NUM_HASH), lambda h: (0, 0)),
        ],
        out_specs=[
            pl.BlockSpec((_S // _CH, _CH), lambda h: (h, 0)),
            pl.BlockSpec((_S // _CH, _CH), lambda h: (h, 0)),
        ],
        out_shape=out_types,
    )(q0, k0, R)


def _pack_body(q_ref, k_ref, v_ref, qpad_ref, kv_ref):
    qb = q_ref[0]
    qpad_ref[0, :, :_D] = qb
    qpad_ref[0, :, _D:] = jnp.zeros_like(qb)
    kv_ref[0, :, :_D] = k_ref[0]
    kv_ref[0, :, _D:] = v_ref[0]


def _pack(q0, k0, v0):
    out_types = (
        jax.ShapeDtypeStruct((_H, _S, _DP), jnp.float32),
        jax.ShapeDtypeStruct((_H, _S, _DP), jnp.float32),
    )
    qpad, kv = pl.pallas_call(
        _pack_body,
        grid=(_H,),
        in_specs=[
            pl.BlockSpec((1, _S, _D), lambda h: (h, 0, 0)),
            pl.BlockSpec((1, _S, _D), lambda h: (h, 0, 0)),
            pl.BlockSpec((1, _S, _D), lambda h: (h, 0, 0)),
        ],
        out_specs=[
            pl.BlockSpec((1, _S, _DP), lambda h: (h, 0, 0)),
            pl.BlockSpec((1, _S, _DP), lambda h: (h, 0, 0)),
        ],
        out_shape=out_types,
    )(q0, k0, v0)
    return qpad.reshape(_ROWS, _DP), kv.reshape(_ROWS, _DP)


_NCODES = 1 << NUM_HASH
_NSORT = 2 * _H


_L = 16
_CPL = _S // _L


def _lg(ref, addr):
    if len(ref.shape) == 2:
        v = plsc.load_gather(ref, [addr >> 7, addr & 127])
        return v.astype(jnp.int32) if v.dtype == jnp.float32 else v
    return plsc.load_gather(ref, [addr])


def _ss(ref, addr, val):
    if ref.dtype == jnp.float32 and val.dtype == jnp.int32:
        val = plsc.bitcast(val, jnp.float32)
    if len(ref.shape) == 2:
        plsc.store_scatter(ref, [addr >> 7, addr & 127], val)
    else:
        plsc.store_scatter(ref, [addr], val)


def _sort_codes(qh, kh):
    @functools.partial(
        pl.kernel,
        out_type=jax.ShapeDtypeStruct((2 * _ROWS // _CH, _CH), jnp.float32),
        scratch_types=[
            pltpu.VMEM((_S // _CH, _CH), jnp.float32),
            pltpu.VMEM((_S,), jnp.int32),
            pltpu.VMEM((_S,), jnp.int32),
            pltpu.VMEM((_S // _CH, _CH), jnp.float32),
            pltpu.VMEM((256 * _L,), jnp.int32),
        ],
        mesh=_sc_mesh(),
        compiler_params=pltpu.CompilerParams(needs_layout_passes=False),
    )
    def k(qh_hbm, kh_hbm, perm_hbm, c0, c1, v1, v2, hist):
        w = _wid()

        @pl.when(w < _H)
        def _():
            pltpu.sync_copy(qh_hbm.at[pl.ds(w * (_S // _CH), _S // _CH)], c0)

        @pl.when((w >= _H) & (w < _NSORT))
        def _():
            pltpu.sync_copy(
                kh_hbm.at[pl.ds((w - _H) * (_S // _CH), _S // _CH)], c0)

        @pl.when(w < _NSORT)
        def _():
            lane = jax.lax.iota(jnp.int32, 16)
            zeros = jnp.zeros((16,), jnp.int32)

            def radix_pass(src_c, src_v, dst_c, dst_v, shift, base):
                def zb(j, c):
                    hist[pl.ds(j * 16, 16)] = zeros
                    return c
                lax.fori_loop(0, 256, zb, 0)

                def hb(j, c):
                    addr = lane * _CPL + j
                    cv = _lg(src_c, addr)
                    digit = (cv >> shift) & 255
                    haddr = digit * _L + lane
                    cur = plsc.load_gather(hist, [haddr])
                    plsc.store_scatter(hist, [haddr], cur + 1)
                    return c
                lax.fori_loop(0, _CPL, hb, 0)

                def sb(j, carry):
                    vec = hist[pl.ds(j * 16, 16)]
                    total = jnp.sum(vec)
                    hist[pl.ds(j * 16, 16)] = plsc.cumsum(vec) - vec + carry
                    return carry + total
                lax.fori_loop(0, 256, sb, jnp.int32(0))

                def pb(j, c):
                    addr = lane * _CPL + j
                    cv = _lg(src_c, addr)
                    vv = addr if src_v is None else plsc.load_gather(
                        src_v, [addr])
                    digit = (cv >> shift) & 255
                    haddr = digit * _L + lane
                    pos = plsc.load_gather(hist, [haddr])
                    plsc.store_scatter(hist, [haddr], pos + 1)
                    if dst_c is not None:
                        plsc.store_scatter(dst_c, [pos], cv)
                    _ss(dst_v, pos, vv + base)
                    return c
                lax.fori_loop(0, _CPL, pb, 0)

            radix_pass(c0, None, c1, v1, 0, 0)
            radix_pass(c1, v1, None, v2, 8, (w % _H) * _S)
            row0 = jnp.where(w < _H, w * (_S // _CH),
                             _ROWS // _CH + (w - _H) * (_S // _CH))
            pltpu.sync_copy(v2, perm_hbm.at[pl.ds(row0, _S // _CH)])

    return k(qh, kh)


_NSLOT = 4
_NGRP = _NCH // _NSLOT


def _bitcast_rows(src_f, dst_i, nch):
    lane = jax.lax.iota(jnp.int32, 16)

    def body(t, c):
        a = t * 16 + lane
        v = plsc.load_gather(src_f, [a >> 7, a & 127])
        plsc.store_scatter(dst_i, [a >> 7, a & 127],
                           plsc.bitcast(v, jnp.int32))
        return c
    lax.fori_loop(0, nch * _CH // 16, body, 0)


_HHALF = _H // 2
_ROWS_HF = _ROWS // 2
_RPW_HF = _ROWS_HF // _NW
_NCH_HF = _RPW_HF // _CH
_PR_HF = _ROWS_HF // _CH


def _ring(gstart, gwait, wstart, wwait, nch):
    ngrp = nch // _NSLOT
    for t in range(_NSLOT):
        gstart(t, t)

    def body(g, carry):
        for t in range(_NSLOT):
            jprev = (g - 1) * _NSLOT + t
            gwait(jprev, t)
            wstart(jprev, t)
        for t in range(_NSLOT):
            j = g * _NSLOT + t
            wwait(j - _NSLOT, t)
            gstart(j, t)
        return carry
    lax.fori_loop(1, ngrp, body, 0)

    for t in range(_NSLOT):
        jlast = (ngrp - 1) * _NSLOT + t
        gwait(jlast, t)
        wstart(jlast, t)
    for t in range(_NSLOT):
        jlast = (ngrp - 1) * _NSLOT + t
        wwait(jlast, t)


def _gather_rows(qpad, kv, perm2d, half):
    out_t = jax.ShapeDtypeStruct((_ROWS_HF, _DP), jnp.float32)

    @functools.partial(
        pl.kernel,
        out_type=(out_t, out_t),
        scratch_types=[
            pltpu.VMEM((_NCH, _CH), jnp.int32),
            pltpu.VMEM((_NCH, _CH), jnp.float32),
            pltpu.VMEM((_NSLOT, _CH, _DP), jnp.float32),
        ] + [pltpu.SemaphoreType.DMA] * (2 * _NSLOT),
        mesh=_sc_mesh(),
        compiler_params=pltpu.CompilerParams(needs_layout_passes=False),
    )
    def k(qf, kvf, perm, qs, kvs, idx_v, idx_f, rows4, *sems):
        gsems, wsems = sems[:_NSLOT], sems[_NSLOT:]
        w = _wid()
        nwh = _NW // 2

        def run_table(tab, wloc, perm_base, out):
            pltpu.sync_copy(
                perm.at[pl.ds(perm_base + wloc * _NCH, _NCH)], idx_f)
            _bitcast_rows(idx_f, idx_v, _NCH)

            def gstart(j, t):
                pltpu.async_copy(tab.at[idx_v.at[j]], rows4.at[t], gsems[t])

            def gwait(j, t):
                pltpu.make_async_copy(
                    tab.at[idx_v.at[j]], rows4.at[t], gsems[t]).wait()

            def wslice(j):
                return out.at[pl.ds(wloc * _RPW + j * _CH, _CH)]

            def wstart(j, t):
                pltpu.async_copy(rows4.at[t], wslice(j), wsems[t])

            def wwait(j, t):
                pltpu.make_async_copy(rows4.at[t], wslice(j), wsems[t]).wait()

            _ring(gstart, gwait, wstart, wwait, _NCH)

        @pl.when(w < nwh)
        def _():
            run_table(qf, w, half * _PR_HF, qs)

        @pl.when(w >= nwh)
        def _():
            run_table(kvf, w - nwh, _ROWS // _CH + half * _PR_HF, kvs)

    return k(qpad, kv, perm2d)


def _scatter_rows(rows_sorted, perm2d):
    @functools.partial(
        pl.kernel,
        out_type=jax.ShapeDtypeStruct((_ROWS, _DP), jnp.float32),
        scratch_types=[
            pltpu.VMEM((_NCH, _CH), jnp.int32),
            pltpu.VMEM((_NCH, _CH), jnp.float32),
            pltpu.VMEM((_NSLOT, _CH, _DP), jnp.float32),
        ] + [pltpu.SemaphoreType.DMA] * (2 * _NSLOT),
        mesh=_sc_mesh(),
        compiler_params=pltpu.CompilerParams(needs_layout_passes=False),
    )
    def k(src0, src1, perm, out, idx_v, idx_f, rows4, *sems):
        rsems, wsems = sems[:_NSLOT], sems[_NSLOT:]
        w = _wid()
        pltpu.sync_copy(perm.at[pl.ds(w * _NCH, _NCH)], idx_f)
        _bitcast_rows(idx_f, idx_v, _NCH)

        def run_half(src, wloc):
            def rslice(j):
                return src.at[pl.ds(wloc * _RPW + j * _CH, _CH)]

            def rstart(j, t):
                pltpu.async_copy(rslice(j), rows4.at[t], rsems[t])

            def rwait(j, t):
                pltpu.make_async_copy(rslice(j), rows4.at[t], rsems[t]).wait()

            def wstart(j, t):
                pltpu.async_copy(rows4.at[t], out.at[idx_v.at[j]], wsems[t])

            def wwait(j, t):
                pltpu.make_async_copy(
                    rows4.at[t], out.at[idx_v.at[j]], wsems[t]).wait()

            _ring(rstart, rwait, wstart, wwait, _NCH)

        @pl.when(w < _NW // 2)
        def _():
            run_half(src0, w)

        @pl.when(w >= _NW // 2)
        def _():
            run_half(src1, w - _NW // 2)

    return k(rows_sorted[0], rows_sorted[1], perm2d)


def _attn_body(qp_ref, kv_ref, samp_ref, out_ref, *, scale, n_over_m):
    for hh in range(_HB):
        qb = qp_ref[hh][:, :_D]
        kb = kv_ref[hh][:, :_D]
        vb = kv_ref[hh][:, _D:]
        ks = samp_ref[hh][:, :_D]
        vs = samp_ref[hh][:, _D:]

        s1 = jax.lax.dot_general(qb, kb, (((1,), (1,)), ((), ())),
                                 preferred_element_type=jnp.float32) * scale
        p1 = jnp.exp(s1)
        l1 = jnp.sum(p1, axis=-1)
        o1 = jax.lax.dot_general(p1, vb, (((1,), (0,)), ((), ())),
                                 preferred_element_type=jnp.float32)

        s2 = jax.lax.dot_general(qb, ks, (((1,), (1,)), ((), ())),
                                 preferred_element_type=jnp.float32) * scale
        p2 = jnp.exp(s2)
        l2 = jnp.sum(p2, axis=-1)
        o2 = jax.lax.dot_general(p2, vs, (((1,), (0,)), ((), ())),
                                 preferred_element_type=jnp.float32)

        den = l1 + n_over_m * l2
        out_ref[hh, :, :_D] = (o1 + n_over_m * o2) / den[:, None]
        out_ref[hh, :, _D:] = jnp.zeros((qb.shape[0], _DP - _D), jnp.float32)


def _fused_attention(qs_pad, kvs, samp):
    bs = BLOCK_SIZE
    nb = _S // bs
    nh = qs_pad.shape[0]
    m = samp.shape[1]
    scale = 1.0 / (_D ** 0.5)
    n_over_m = float(_S) / float(m)

    body = functools.partial(_attn_body, scale=scale, n_over_m=n_over_m)
    return pl.pallas_call(
        body,
        grid=(nh // _HB, nb),
        in_specs=[
            pl.BlockSpec((_HB, bs, _DP), lambda h, i: (h, i, 0)),
            pl.BlockSpec((_HB, bs, _DP), lambda h, i: (h, i, 0)),
            pl.BlockSpec((_HB, m, _DP), lambda h, i: (h, 0, 0)),
        ],
        out_specs=pl.BlockSpec((_HB, bs, _DP), lambda h, i: (h, i, 0)),
        out_shape=jax.ShapeDtypeStruct((nh, _S, _DP), jnp.float32),
    )(qs_pad, kvs, samp)


def kernel(q, k, v, R):
    B, H, S, D = q.shape
    assert (B, H, S, D) == (1, _H, _S, _D)

    q0, k0, v0 = q[0], k[0], v[0]
    qh2d, kh2d = _hash(q0, k0, R)
    perm2d = _sort_codes(qh2d, kh2d)
    qpad, kv = _pack(q0, k0, v0)

    stride = _S // SAMPLE_SIZE
    samp = kv.reshape(_H, _S, _DP)[:, ::stride, :]

    halves = []
    gathered = [_gather_rows(qpad, kv, perm2d, hf) for hf in (0, 1)]
    for hf, (qsf, kvsf) in enumerate(gathered):
        qs_pad = qsf.reshape(_HHALF, _S, _DP)
        kvs = kvsf.reshape(_HHALF, _S, _DP)
        s = samp[hf * _HHALF:(hf + 1) * _HHALF]
        halves.append(_fused_attention(qs_pad, kvs, s).reshape(
            _ROWS_HF, _DP))

    outf = _scatter_rows((halves[0], halves[1]), perm2d)
    return outf[:, :_D].reshape(1, _H, _S, _D)
